# baseline (speedup 1.0000x reference)
import functools

import jax
import jax.numpy as jnp
from jax import lax
from jax.experimental import pallas as pl
from jax.experimental.pallas import tpu as pltpu
from jax.experimental.pallas import tpu_sc as plsc

N_NODES = 10000
N_EDGES = 320000
D_FEAT = 128
D_EDGE = 16
N_GRAPHS = 64
D_U = 64
D_HID = 128
D_OUT = 128

NC = 2
NS = 16
NW = NC * NS
IDX_W = 128
E_PER_W = 10240
E_TAIL = N_EDGES - (NW - 1) * E_PER_W
CHUNK = 1024
SCAT_PER_CHUNK = CHUNK // IDX_W
N_CHUNKS = E_PER_W // CHUNK
N_PAD = 10240
ROWS_PER_TILE = N_PAD // NS

@functools.cache
def _make_sc_segment_sum():
    mesh = plsc.VectorSubcoreMesh(core_axis_name="c", subcore_axis_name="s")
    return pl.kernel(
        _sc_segment_sum_body,
        mesh=mesh,
        out_type=jax.ShapeDtypeStruct((NC, N_PAD // 8, 8, 128), jnp.float32),
        scratch_types=[
            pltpu.VMEM((E_PER_W // 128, 128), jnp.int32),
            pltpu.VMEM((4, CHUNK // 128, 8, 128), jnp.float32),
            pltpu.VMEM((3, CHUNK, D_EDGE), jnp.float32),
            pltpu.VMEM_SHARED((N_PAD, D_EDGE), jnp.float32),
            pltpu.SemaphoreType.DMA,
            pltpu.SemaphoreType.DMA,
            pltpu.SemaphoreType.DMA,
            pltpu.SemaphoreType.DMA,
            pltpu.SemaphoreType.DMA,
            pltpu.SemaphoreType.DMA,
        ],
        compiler_params=pltpu.CompilerParams(use_tc_tiling_on_sc=False,
                                             needs_layout_passes=False),
    )


def _sc_segment_sum_body(eidx_hbm, eaT_hbm, out_hbm, idx_v, cols_v, rows_v,
                         acc_sh, sem_ix, sem_l0, sem_l1, sem_s0, sem_s1,
                         sem_s2):
    cid = lax.axis_index("c")
    sid = lax.axis_index("s")
    wid = sid * NC + cid
    wbase = wid * E_PER_W
    lanes = lax.iota(jnp.int32, 16)
    sem_l = (sem_l0, sem_l1)
    sem_s = (sem_s0, sem_s1, sem_s2)

    def _load(c, n_tiles):
        b = c % 2
        t0 = (wbase + c * CHUNK) // 128
        return [pltpu.async_copy(eaT_hbm.at[fh, pl.ds(t0, n_tiles)],
                                 cols_v.at[b * 2 + fh, pl.ds(0, n_tiles)],
                                 sem_l[b])
                for fh in range(2)]

    def _xpose_chunk(c, nscat):
        b = c % 2
        rb = c % 3

        @plsc.parallel_loop(0, nscat * IDX_W // 16, unroll=2)
        def _xpose(g):
            t = g // 8
            l0 = (g % 8) * 16
            rid = g * 16 + lanes
            for fh in range(2):
                for fl in range(8):
                    vec = cols_v[b * 2 + fh, t, fl, pl.ds(l0, 16)]
                    plsc.store_scatter(
                        rows_v.at[rb],
                        [rid, jnp.full((16,), fh * 8 + fl, jnp.int32)], vec)

    def _fire_scatters(c, nscat):
        rb = c % 3
        return [pltpu.async_copy(rows_v.at[rb, pl.ds(j * IDX_W, IDX_W)],
                                 acc_sh.at[idx_v.at[c * SCAT_PER_CHUNK + j]],
                                 sem_s[rb], add=True)
                for j in range(nscat)]

    def _run(n_full, n_last_scat):
        nscats = [SCAT_PER_CHUNK] * n_full + [n_last_scat]
        n = len(nscats)
        n_idx_rows = sum(nscats)
        ix_desc = pltpu.async_copy(
            eidx_hbm.at[pl.ds(wbase // 128, n_idx_rows), 0],
            idx_v.at[pl.ds(0, n_idx_rows)], sem_ix)

        load_descs = {0: _load(0, nscats[0])}

        @plsc.parallel_loop(0, ROWS_PER_TILE, unroll=4)
        def _zero_row(i):
            rows_v[0, i, :] = jnp.zeros((D_EDGE,), jnp.float32)
        pltpu.sync_copy(rows_v.at[0, pl.ds(0, ROWS_PER_TILE)],
                        acc_sh.at[pl.ds(sid * ROWS_PER_TILE, ROWS_PER_TILE)])
        ix_desc.wait()
        plsc.subcore_barrier()

        scat_descs = {}
        for c in range(n):
            for d in load_descs.pop(c):
                d.wait()
            if c + 1 < n:
                load_descs[c + 1] = _load(c + 1, nscats[c + 1])
            if c - 3 in scat_descs:
                for d in scat_descs.pop(c - 3):
                    d.wait()
            _xpose_chunk(c, nscats[c])
            scat_descs[c] = _fire_scatters(c, nscats[c])
        for descs in scat_descs.values():
            for d in descs:
                d.wait()

    @pl.when(wid < NW - 1)
    def _full_worker():
        _run(N_CHUNKS - 1, SCAT_PER_CHUNK)

    @pl.when(wid == NW - 1)
    def _tail_worker():
        _run(E_TAIL // CHUNK, (E_TAIL % CHUNK) // IDX_W)

    plsc.subcore_barrier()
    out_descs = [
        pltpu.async_copy(
            acc_sh.at[pl.ds((sid * ROWS_PER_TILE // 8 + k) * 8, 8)],
            out_hbm.at[cid, sid * (ROWS_PER_TILE // 8) + k, :, pl.ds(0, 16)],
            sem_ix)
        for k in range(ROWS_PER_TILE // 8)]
    for d in out_descs:
        d.wait()


NB = 1000
N_BLOCKS = N_NODES // NB


def _tc_pre_body(x_ref, v_ref, u_ref, w1_ref, b1_ref, t_ref):
    v = v_ref[0, 0, :]
    iota = lax.broadcasted_iota(jnp.int32, (NB, N_GRAPHS), 1)
    onehot = (v[:, None] == iota).astype(jnp.float32)
    m = jnp.dot(u_ref[...], w1_ref[D_FEAT + D_EDGE:, :],
                preferred_element_type=jnp.float32)
    t_ref[...] = (jnp.dot(x_ref[...], w1_ref[:D_FEAT, :],
                          preferred_element_type=jnp.float32)
                  + jnp.dot(onehot, m, preferred_element_type=jnp.float32)
                  + b1_ref[...])


_tc_pre = pl.pallas_call(
    _tc_pre_body,
    grid=(N_BLOCKS,),
    in_specs=[
        pl.BlockSpec((NB, D_FEAT), lambda i: (i, 0)),
        pl.BlockSpec((1, 1, NB), lambda i: (i, 0, 0)),
        pl.BlockSpec((N_GRAPHS, D_U), lambda i: (0, 0)),
        pl.BlockSpec((D_FEAT + D_EDGE + D_U, D_HID), lambda i: (0, 0)),
        pl.BlockSpec((1, D_HID), lambda i: (0, 0)),
    ],
    out_specs=pl.BlockSpec((NB, D_HID), lambda i: (i, 0)),
    out_shape=jax.ShapeDtypeStruct((N_NODES, D_HID), jnp.float32),
    compiler_params=pltpu.CompilerParams(
        dimension_semantics=("parallel",)),
)


def _tc_post_body(t_ref, p_ref, w1_ref, w2_ref, b2_ref, o_ref):
    packed = p_ref[0] + p_ref[1]
    agg = packed[:, :, :D_EDGE].reshape(NB, D_EDGE)
    h = t_ref[...] + jnp.dot(agg, w1_ref[D_FEAT:D_FEAT + D_EDGE, :],
                             preferred_element_type=jnp.float32)
    h = jnp.maximum(h, 0.0)
    o_ref[...] = (jnp.dot(h, w2_ref[...], preferred_element_type=jnp.float32)
                  + b2_ref[...])


_tc_post = pl.pallas_call(
    _tc_post_body,
    grid=(N_BLOCKS,),
    in_specs=[
        pl.BlockSpec((NB, D_HID), lambda i: (i, 0)),
        pl.BlockSpec((NC, NB // 8, 8, 128), lambda i: (0, i, 0, 0)),
        pl.BlockSpec((D_FEAT + D_EDGE + D_U, D_HID), lambda i: (0, 0)),
        pl.BlockSpec((D_HID, D_OUT), lambda i: (0, 0)),
        pl.BlockSpec((1, D_OUT), lambda i: (0, 0)),
    ],
    out_specs=pl.BlockSpec((NB, D_OUT), lambda i: (i, 0)),
    out_shape=jax.ShapeDtypeStruct((N_NODES, D_OUT), jnp.float32),
    compiler_params=pltpu.CompilerParams(
        dimension_semantics=("parallel",)),
)


def kernel(x, edge_index, edge_attr, u, v_indices, W1, b1, W2, b2):
    ea4 = edge_attr.T.reshape(2, 8, N_EDGES // 128, 128).transpose(0, 2, 1, 3)
    ei3 = edge_index.astype(jnp.int32).reshape(2, N_EDGES // 128,
                                               128).transpose(1, 0, 2)
    partials = _make_sc_segment_sum()(ei3, ea4)
    v3d = v_indices.astype(jnp.int32).reshape(N_BLOCKS, 1, NB)
    t = _tc_pre(x, v3d, u, W1, b1.reshape(1, D_HID))
    return _tc_post(t, partials, W1, W2, b2.reshape(1, D_OUT))


# speedup vs baseline: 1.0838x; 1.0838x over previous
import functools

import jax
import jax.numpy as jnp
from jax import lax
from jax.experimental import pallas as pl
from jax.experimental.pallas import tpu as pltpu
from jax.experimental.pallas import tpu_sc as plsc

N_NODES = 10000
N_EDGES = 320000
D_FEAT = 128
D_EDGE = 16
N_GRAPHS = 64
D_U = 64
D_HID = 128
D_OUT = 128

NC = 2
NS = 16
NW = NC * NS
IDX_W = 128
E_PER_W = 10240
E_TAIL = N_EDGES - (NW - 1) * E_PER_W
CHUNK = 1024
SCAT_PER_CHUNK = CHUNK // IDX_W
N_CHUNKS = E_PER_W // CHUNK
N_PAD = 10240
ROWS_PER_TILE = N_PAD // NS

@functools.cache
def _make_sc_segment_sum():
    mesh = plsc.VectorSubcoreMesh(core_axis_name="c", subcore_axis_name="s")
    return pl.kernel(
        _sc_segment_sum_body,
        mesh=mesh,
        out_type=jax.ShapeDtypeStruct((NC, N_PAD // 1024, 128, 128),
                                      jnp.float32),
        scratch_types=[
            pltpu.VMEM((E_PER_W // 128, 128), jnp.int32),
            pltpu.VMEM((4, CHUNK // 128, 8, 128), jnp.float32),
            pltpu.VMEM((3, CHUNK, D_EDGE), jnp.float32),
            pltpu.VMEM_SHARED((N_PAD, D_EDGE), jnp.float32),
            pltpu.SemaphoreType.DMA,
            pltpu.SemaphoreType.DMA,
            pltpu.SemaphoreType.DMA,
            pltpu.SemaphoreType.DMA,
            pltpu.SemaphoreType.DMA,
            pltpu.SemaphoreType.DMA,
        ],
        compiler_params=pltpu.CompilerParams(use_tc_tiling_on_sc=False,
                                             needs_layout_passes=False),
    )


def _sc_segment_sum_body(eidx_hbm, eaT_hbm, out_hbm, idx_v, cols_v, rows_v,
                         acc_sh, sem_ix, sem_l0, sem_l1, sem_s0, sem_s1,
                         sem_s2):
    cid = lax.axis_index("c")
    sid = lax.axis_index("s")
    wid = sid * NC + cid
    wbase = wid * E_PER_W
    lanes = lax.iota(jnp.int32, 16)
    sem_l = (sem_l0, sem_l1)
    sem_s = (sem_s0, sem_s1, sem_s2)

    def _load(c, n_tiles):
        b = c % 2
        t0 = (wbase + c * CHUNK) // 128
        return [pltpu.async_copy(eaT_hbm.at[fh, pl.ds(t0, n_tiles)],
                                 cols_v.at[b * 2 + fh, pl.ds(0, n_tiles)],
                                 sem_l[b])
                for fh in range(2)]

    def _xpose_chunk(c, nscat):
        b = c % 2
        rb = c % 3

        @plsc.parallel_loop(0, nscat * IDX_W // 16, unroll=2)
        def _xpose(g):
            t = g // 8
            l0 = (g % 8) * 16
            rid = g * 16 + lanes
            for fh in range(2):
                for fl in range(8):
                    vec = cols_v[b * 2 + fh, t, fl, pl.ds(l0, 16)]
                    plsc.store_scatter(
                        rows_v.at[rb],
                        [rid, jnp.full((16,), fh * 8 + fl, jnp.int32)], vec)

    def _fire_scatters(c, nscat):
        rb = c % 3
        return [pltpu.async_copy(rows_v.at[rb, pl.ds(j * IDX_W, IDX_W)],
                                 acc_sh.at[idx_v.at[c * SCAT_PER_CHUNK + j]],
                                 sem_s[rb], add=True)
                for j in range(nscat)]

    def _run(n_full, n_last_scat):
        nscats = [SCAT_PER_CHUNK] * n_full + [n_last_scat]
        n = len(nscats)
        n_idx_rows = sum(nscats)
        ix_desc = pltpu.async_copy(
            eidx_hbm.at[pl.ds(wbase // 128, n_idx_rows), 0],
            idx_v.at[pl.ds(0, n_idx_rows)], sem_ix)

        load_descs = {0: _load(0, nscats[0])}

        @plsc.parallel_loop(0, ROWS_PER_TILE, unroll=4)
        def _zero_row(i):
            rows_v[0, i, :] = jnp.zeros((D_EDGE,), jnp.float32)
        pltpu.sync_copy(rows_v.at[0, pl.ds(0, ROWS_PER_TILE)],
                        acc_sh.at[pl.ds(sid * ROWS_PER_TILE, ROWS_PER_TILE)])
        ix_desc.wait()
        plsc.subcore_barrier()

        scat_descs = {}
        for c in range(n):
            for d in load_descs.pop(c):
                d.wait()
            if c + 1 < n:
                load_descs[c + 1] = _load(c + 1, nscats[c + 1])
            if c - 3 in scat_descs:
                for d in scat_descs.pop(c - 3):
                    d.wait()
            _xpose_chunk(c, nscats[c])
            scat_descs[c] = _fire_scatters(c, nscats[c])
        for descs in scat_descs.values():
            for d in descs:
                d.wait()

    @pl.when(wid < NW - 1)
    def _full_worker():
        _run(N_CHUNKS - 1, SCAT_PER_CHUNK)

    @pl.when(wid == NW - 1)
    def _tail_worker():
        _run(E_TAIL // CHUNK, (E_TAIL % CHUNK) // IDX_W)

    plsc.subcore_barrier()
    out_descs = []
    for seg in range(ROWS_PER_TILE // 128):
        n0 = sid * ROWS_PER_TILE + seg * 128
        blk = n0 // 1024
        k = (n0 // 128) % 8
        out_descs.append(pltpu.async_copy(
            acc_sh.at[pl.ds(n0, 128)],
            out_hbm.at[cid, blk, :, pl.ds(k * 16, 16)], sem_ix))
    for d in out_descs:
        d.wait()


NB = 1024
N_BLOCKS = 10


def _tc_pre_body(x_ref, v_ref, u_ref, w1_ref, b1_ref, t_ref):
    v = v_ref[0, 0, :]
    iota = lax.broadcasted_iota(jnp.int32, (NB, N_GRAPHS), 1)
    onehot = (v[:, None] == iota).astype(jnp.float32)
    m = jnp.dot(u_ref[...], w1_ref[D_FEAT + D_EDGE:, :],
                preferred_element_type=jnp.float32)
    t_ref[...] = (jnp.dot(x_ref[...], w1_ref[:D_FEAT, :],
                          preferred_element_type=jnp.float32)
                  + jnp.dot(onehot, m, preferred_element_type=jnp.float32)
                  + b1_ref[...])


_tc_pre = pl.pallas_call(
    _tc_pre_body,
    grid=(N_BLOCKS,),
    in_specs=[
        pl.BlockSpec((NB, D_FEAT), lambda i: (i, 0)),
        pl.BlockSpec((1, 1, NB), lambda i: (i, 0, 0)),
        pl.BlockSpec((N_GRAPHS, D_U), lambda i: (0, 0)),
        pl.BlockSpec((D_FEAT + D_EDGE + D_U, D_HID), lambda i: (0, 0)),
        pl.BlockSpec((1, D_HID), lambda i: (0, 0)),
    ],
    out_specs=pl.BlockSpec((NB, D_HID), lambda i: (i, 0)),
    out_shape=jax.ShapeDtypeStruct((N_NODES, D_HID), jnp.float32),
    compiler_params=pltpu.CompilerParams(
        dimension_semantics=("parallel",)),
)


def _tc_post_body(t_ref, p_ref, w1_ref, w2_ref, b2_ref, o_ref):
    packed = p_ref[0, 0] + p_ref[1, 0]
    agg = jnp.concatenate(
        [packed[:, 16 * k:16 * (k + 1)] for k in range(8)], axis=0)
    h = t_ref[...] + jnp.dot(agg, w1_ref[D_FEAT:D_FEAT + D_EDGE, :],
                             preferred_element_type=jnp.float32)
    h = jnp.maximum(h, 0.0)
    o_ref[...] = (jnp.dot(h, w2_ref[...], preferred_element_type=jnp.float32)
                  + b2_ref[...])


_tc_post = pl.pallas_call(
    _tc_post_body,
    grid=(N_BLOCKS,),
    in_specs=[
        pl.BlockSpec((NB, D_HID), lambda i: (i, 0)),
        pl.BlockSpec((NC, 1, 128, 128), lambda i: (0, i, 0, 0)),
        pl.BlockSpec((D_FEAT + D_EDGE + D_U, D_HID), lambda i: (0, 0)),
        pl.BlockSpec((D_HID, D_OUT), lambda i: (0, 0)),
        pl.BlockSpec((1, D_OUT), lambda i: (0, 0)),
    ],
    out_specs=pl.BlockSpec((NB, D_OUT), lambda i: (i, 0)),
    out_shape=jax.ShapeDtypeStruct((N_NODES, D_OUT), jnp.float32),
    compiler_params=pltpu.CompilerParams(
        dimension_semantics=("parallel",)),
)


def kernel(x, edge_index, edge_attr, u, v_indices, W1, b1, W2, b2):
    ea4 = edge_attr.T.reshape(2, 8, N_EDGES // 128, 128).transpose(0, 2, 1, 3)
    ei3 = edge_index.astype(jnp.int32).reshape(2, N_EDGES // 128,
                                               128).transpose(1, 0, 2)
    partials = _make_sc_segment_sum()(ei3, ea4)
    v_pad = jnp.pad(v_indices.astype(jnp.int32),
                    (0, N_BLOCKS * NB - N_NODES))
    v3d = v_pad.reshape(N_BLOCKS, 1, NB)
    t = _tc_pre(x, v3d, u, W1, b1.reshape(1, D_HID))
    return _tc_post(t, partials, W1, W2, b2.reshape(1, D_OUT))
